# baseline (speedup 1.0000x reference)
import functools

import jax
import jax.numpy as jnp
from jax import lax
from jax.experimental import pallas as pl
from jax.experimental.pallas import tpu as pltpu
from jax.experimental.pallas import tpu_sc as plsc

EMB = 32
B = 4096
L = 50
LPAD = 64
NW = 32
ROWS = 2 * B
ROWS_PER_W = ROWS // NW
STEPS = ROWS_PER_W * LPAD // 128
IDX_PER_W = STEPS * 128
ADDR_PER_W = IDX_PER_W * 16


def _make_pool_kernel():
    mesh = plsc.VectorSubcoreMesh(core_axis_name="c", subcore_axis_name="s")

    @functools.partial(
        pl.kernel,
        mesh=mesh,
        out_type=jax.ShapeDtypeStruct((ROWS, EMB), jnp.float32),
        scratch_types=[
            pltpu.VMEM((STEPS, 128), jnp.int32),
            pltpu.VMEM((2, 128 * 16), jnp.int32),
            pltpu.VMEM((2, 128, 128), jnp.float32),
            pltpu.VMEM((ROWS_PER_W, EMB), jnp.float32),
            pltpu.SemaphoreType.DMA,
            pltpu.SemaphoreType.DMA,
            pltpu.SemaphoreType.DMA,
            pltpu.SemaphoreType.DMA,
        ],
        compiler_params=pltpu.CompilerParams(needs_layout_passes=False),
    )
    def pool(table_hbm, ridx_hbm, addr_hbm, out_hbm,
             idx_v, addr_v, rows_v, out_v, semg0, semg1, sema0, sema1):
        wid = lax.axis_index("s") * 2 + lax.axis_index("c")
        pltpu.sync_copy(ridx_hbm.at[pl.ds(wid * STEPS, STEPS), :], idx_v)
        semg = (semg0, semg1)
        sema = (sema0, sema1)

        def issue(st, b):
            pltpu.async_copy(table_hbm.at[idx_v.at[st]],
                             rows_v.at[b], semg[b])
            pltpu.async_copy(
                addr_hbm.at[pl.ds(wid * ADDR_PER_W + st * 2048, 2048)],
                addr_v.at[b], sema[b])

        def wait(b):
            pltpu.make_async_copy(table_hbm.at[idx_v.at[0]],
                                  rows_v.at[b], semg[b]).wait()
            pltpu.make_async_copy(addr_hbm.at[pl.ds(0, 2048)],
                                  addr_v.at[b], sema[b]).wait()

        def accumulate(st, b):
            rowbuf = rows_v.at[b]
            for half in range(2):
                accs = [jnp.zeros((16,), jnp.float32) for _ in range(8)]
                for j in range(LPAD):
                    e = half * LPAD + j
                    rowv = jnp.full((16,), e, jnp.int32)
                    av = addr_v[b, pl.ds(e * 16, 16)]
                    v0 = plsc.load_gather(rowbuf, [rowv, av])
                    v1 = plsc.load_gather(rowbuf, [rowv, av + 16])
                    k = j % 4
                    accs[k] = accs[k] + v0
                    accs[4 + k] = accs[4 + k] + v1
                s = st * 2 + half
                out_v[s, pl.ds(0, 16)] = (accs[0] + accs[1]) + (accs[2] + accs[3])
                out_v[s, pl.ds(16, 16)] = (accs[4] + accs[5]) + (accs[6] + accs[7])

        issue(0, 0)

        @pl.loop(0, STEPS, step=2)
        def _(st):
            for b in range(2):
                cur = st + b

                @pl.when(cur + 1 < STEPS)
                def _():
                    issue(cur + 1, 1 - b)

                wait(b)
                accumulate(cur, b)

        pltpu.sync_copy(out_v, out_hbm.at[pl.ds(wid * ROWS_PER_W, ROWS_PER_W), :])

    return pool


_pool = _make_pool_kernel()


def _mlp_body(p_ref, il_ref, w1a_ref, w1b_ref, b1_ref, w2_ref, b2_ref, o_ref):
    x0 = p_ref[0] / il_ref[0]
    x1 = p_ref[1] / il_ref[1]
    h = jnp.dot(x0, w1a_ref[...], preferred_element_type=jnp.float32)
    h = h + jnp.dot(x1, w1b_ref[...], preferred_element_type=jnp.float32)
    h = jnp.maximum(h + b1_ref[...], 0.0)
    o_ref[...] = jnp.dot(h, w2_ref[...], preferred_element_type=jnp.float32) + b2_ref[...]


def kernel(data, length, embed_table, W1, b1, W2, b2):
    flat = jnp.pad(data.reshape(ROWS, L), ((0, 0), (0, LPAD - L)))
    ridx2 = (flat >> 2).reshape(NW * STEPS, 128)
    lanevec = jnp.tile(jnp.arange(16, dtype=jnp.int32), 128)
    addr1 = (jnp.repeat((flat & 3) * EMB, 16, axis=-1).reshape(NW * STEPS, 2048)
             + lanevec[None, :]).reshape(-1)
    ctable = embed_table.reshape(-1, 128)
    pooled = _pool(ctable, ridx2, addr1).reshape(2, B, EMB)
    lenf = length.astype(jnp.float32).reshape(2, B, 1)
    w2p = jnp.pad(W2, ((0, 0), (0, 128 - W2.shape[1])))
    b2p = jnp.pad(b2, (0, 128 - b2.shape[0]))
    out = pl.pallas_call(
        _mlp_body,
        out_shape=jax.ShapeDtypeStruct((B, 128), jnp.float32),
    )(pooled, lenf, W1[:EMB], W1[EMB:], b1.reshape(1, -1),
      w2p, b2p.reshape(1, -1))
    return out[:, :3]


# speedup vs baseline: 8.4563x; 8.4563x over previous
import functools

import jax
import jax.numpy as jnp
from jax import lax
from jax.experimental import pallas as pl
from jax.experimental.pallas import tpu as pltpu
from jax.experimental.pallas import tpu_sc as plsc

VOCAB = 1000000
EMB = 32
B = 4096
L = 50
NW = 32
ROWS = 2 * B
ROWS_PER_W = ROWS // NW
CHUNK = 8
N_CHUNKS = ROWS_PER_W // CHUNK
IDX_PER_CHUNK = CHUNK * L


def _accumulate(rows_v, out_v, r):
    for h in (0, 16):
        accs = [jnp.zeros((16,), jnp.float32) for _ in range(4)]
        for j in range(L):
            accs[j % 4] = accs[j % 4] + rows_v[r * L + j, pl.ds(h, 16)]
        out_v[r, pl.ds(h, 16)] = (accs[0] + accs[1]) + (accs[2] + accs[3])


def _make_pool_kernel():
    mesh = plsc.VectorSubcoreMesh(core_axis_name="c", subcore_axis_name="s")

    @functools.partial(
        pl.kernel,
        mesh=mesh,
        out_type=jax.ShapeDtypeStruct((ROWS, EMB), jnp.float32),
        scratch_types=[
            pltpu.VMEM((IDX_PER_CHUNK,), jnp.int32),
            pltpu.VMEM((IDX_PER_CHUNK, EMB), jnp.float32),
            pltpu.VMEM((CHUNK, EMB), jnp.float32),
            pltpu.SemaphoreType.DMA,
        ],
        compiler_params=pltpu.CompilerParams(use_tc_tiling_on_sc=False),
    )
    def pool(table_hbm, idx_hbm, out_hbm, idx_v, rows_v, out_v, sem):
        wid = lax.axis_index("s") * 2 + lax.axis_index("c")
        base_row = wid * ROWS_PER_W

        @pl.loop(0, N_CHUNKS)
        def _(ci):
            row0 = base_row + ci * CHUNK
            pltpu.sync_copy(idx_hbm.at[pl.ds(row0 * L, IDX_PER_CHUNK)], idx_v)
            pltpu.async_copy(table_hbm.at[idx_v], rows_v, sem).wait()
            for r in range(CHUNK):
                _accumulate(rows_v, out_v, r)
            pltpu.sync_copy(out_v, out_hbm.at[pl.ds(row0, CHUNK), :])

    return pool


_pool = _make_pool_kernel()


def _mlp_body(p_ref, il_ref, w1a_ref, w1b_ref, b1_ref, w2_ref, b2_ref, o_ref):
    x0 = p_ref[0] / il_ref[0]
    x1 = p_ref[1] / il_ref[1]
    h = jnp.dot(x0, w1a_ref[...], preferred_element_type=jnp.float32)
    h = h + jnp.dot(x1, w1b_ref[...], preferred_element_type=jnp.float32)
    h = jnp.maximum(h + b1_ref[...], 0.0)
    o_ref[...] = jnp.dot(h, w2_ref[...], preferred_element_type=jnp.float32) + b2_ref[...]


def kernel(data, length, embed_table, W1, b1, W2, b2):
    idx_flat = data.reshape(-1)
    packed = lax.optimization_barrier(embed_table.reshape(-1, 128))
    table = packed.reshape(VOCAB, EMB)
    pooled = _pool(table, idx_flat).reshape(2, B, EMB)
    lenf = length.astype(jnp.float32).reshape(2, B, 1)
    w2p = jnp.pad(W2, ((0, 0), (0, 128 - W2.shape[1])))
    b2p = jnp.pad(b2, (0, 128 - b2.shape[0]))
    out = pl.pallas_call(
        _mlp_body,
        out_shape=jax.ShapeDtypeStruct((B, 128), jnp.float32),
    )(pooled, lenf, W1[:EMB], W1[EMB:], b1.reshape(1, -1),
      w2p, b2p.reshape(1, -1))
    return out[:, :3]


# speedup vs baseline: 8.4668x; 1.0012x over previous
import functools

import jax
import jax.numpy as jnp
from jax import lax
from jax.experimental import pallas as pl
from jax.experimental.pallas import tpu as pltpu
from jax.experimental.pallas import tpu_sc as plsc

VOCAB = 1000000
EMB = 32
B = 4096
L = 50
NW = 32
ROWS = 2 * B
ROWS_PER_W = ROWS // NW
CHUNK = 8
N_CHUNKS = ROWS_PER_W // CHUNK
IDX_PER_CHUNK = CHUNK * L


def _accumulate(rows_v, out_v, r):
    for h in (0, 16):
        accs = [jnp.zeros((16,), jnp.float32) for _ in range(4)]
        for j in range(L):
            accs[j % 4] = accs[j % 4] + rows_v[r * L + j, pl.ds(h, 16)]
        out_v[r, pl.ds(h, 16)] = (accs[0] + accs[1]) + (accs[2] + accs[3])


def _make_pool_kernel():
    mesh = plsc.VectorSubcoreMesh(core_axis_name="c", subcore_axis_name="s")

    @functools.partial(
        pl.kernel,
        mesh=mesh,
        out_type=jax.ShapeDtypeStruct((ROWS, EMB), jnp.float32),
        scratch_types=[
            pltpu.VMEM((IDX_PER_CHUNK,), jnp.int32),
            pltpu.VMEM((IDX_PER_CHUNK, EMB), jnp.float32),
            pltpu.VMEM((CHUNK, EMB), jnp.float32),
            pltpu.SemaphoreType.DMA,
        ],
        compiler_params=pltpu.CompilerParams(use_tc_tiling_on_sc=False),
    )
    def pool(table3_hbm, idx_hbm, out_hbm, idx_v, rows_v, out_v, sem):
        table_hbm = table3_hbm.at[0]
        wid = lax.axis_index("s") * 2 + lax.axis_index("c")
        base_row = wid * ROWS_PER_W

        @pl.loop(0, N_CHUNKS)
        def _(ci):
            row0 = base_row + ci * CHUNK
            pltpu.sync_copy(idx_hbm.at[pl.ds(row0 * L, IDX_PER_CHUNK)], idx_v)
            pltpu.async_copy(table_hbm.at[idx_v], rows_v, sem).wait()
            for r in range(CHUNK):
                _accumulate(rows_v, out_v, r)
            pltpu.sync_copy(out_v, out_hbm.at[pl.ds(row0, CHUNK), :])

    return pool


_pool = _make_pool_kernel()


def _mlp_body(p_ref, il_ref, w1a_ref, w1b_ref, b1_ref, w2_ref, b2_ref, o_ref):
    x0 = p_ref[0] / il_ref[0]
    x1 = p_ref[1] / il_ref[1]
    h = jnp.dot(x0, w1a_ref[...], preferred_element_type=jnp.float32)
    h = h + jnp.dot(x1, w1b_ref[...], preferred_element_type=jnp.float32)
    h = jnp.maximum(h + b1_ref[...], 0.0)
    o_ref[...] = jnp.dot(h, w2_ref[...], preferred_element_type=jnp.float32) + b2_ref[...]


def kernel(data, length, embed_table, W1, b1, W2, b2):
    idx_flat = data.reshape(-1)
    table3 = embed_table.reshape(1, VOCAB, EMB)
    pooled = _pool(table3, idx_flat).reshape(2, B, EMB)
    lenf = length.astype(jnp.float32).reshape(2, B, 1)
    w2p = jnp.pad(W2, ((0, 0), (0, 128 - W2.shape[1])))
    b2p = jnp.pad(b2, (0, 128 - b2.shape[0]))
    out = pl.pallas_call(
        _mlp_body,
        out_shape=jax.ShapeDtypeStruct((B, 128), jnp.float32),
    )(pooled, lenf, W1[:EMB], W1[EMB:], b1.reshape(1, -1),
      w2p, b2p.reshape(1, -1))
    return out[:, :3]
